# baseline (speedup 1.0000x reference)
import jax
import jax.numpy as jnp
from jax import lax
from jax.experimental import pallas as pl
from jax.experimental.pallas import tpu as pltpu
from jax.experimental.pallas import tpu_sc as plsc

_NC = 2
_NS = 16
_L = 16
_HB = 8
_BB = 128


def kernel(token_ids, table):
    B, H = token_ids.shape
    V, D = table.shape
    idx_t = token_ids.T
    tab_flat = table.reshape(-1)

    nw = _NC * _NS
    bblocks_per_w = B // _BB // nw
    hblocks = H // _HB
    nblocks = bblocks_per_w * hblocks

    mesh = plsc.VectorSubcoreMesh(core_axis_name="core", subcore_axis_name="subcore")

    @pl.kernel(
        out_type=jax.ShapeDtypeStruct((H, D, B), jnp.float32),
        mesh=mesh,
        scratch_types=[
            pltpu.VMEM((V * D,), jnp.float32),
            pltpu.VMEM((2, _HB, _BB), jnp.int32),
            pltpu.VMEM((2, _HB, D, _BB), jnp.float32),
            pltpu.SemaphoreType.DMA,
            pltpu.SemaphoreType.DMA,
            pltpu.SemaphoreType.DMA,
        ],
        compiler_params=pltpu.CompilerParams(
            use_tc_tiling_on_sc=True, needs_layout_passes=False
        ),
    )
    def emb_kernel(tab_hbm, idx_hbm, out_hbm, tab_v, idx_v, o_v, sem_t, sem_i, sem_o):
        wid = lax.axis_index("subcore") * _NC + lax.axis_index("core")
        b0 = wid * (bblocks_per_w * _BB)
        pltpu.async_copy(tab_hbm, tab_v, sem_t).wait()

        def blk_slices(blk):
            hb = lax.rem(blk, hblocks)
            bb = blk // hblocks
            return pl.ds(hb * _HB, _HB), pl.ds(b0 + bb * _BB, _BB)

        def idx_dma(blk, buf):
            hs, bs = blk_slices(blk)
            return pltpu.make_async_copy(idx_hbm.at[hs, bs], idx_v.at[buf], sem_i)

        def out_dma(blk, buf):
            hs, bs = blk_slices(blk)
            return pltpu.make_async_copy(
                o_v.at[buf], out_hbm.at[hs, :, bs], sem_o
            )

        idx_dma(0, 0).start()

        @pl.loop(0, nblocks)
        def _(blk):
            buf = lax.rem(blk, 2)
            nxt = 1 - buf

            @pl.when(blk + 1 < nblocks)
            def _():
                idx_dma(blk + 1, nxt).start()

            idx_dma(blk, buf).wait()

            @pl.when(blk >= 2)
            def _():
                out_dma(blk - 2, buf).wait()

            @pl.loop(0, _HB * (_BB // _L))
            def _(q):
                h = q // (_BB // _L)
                bq = lax.rem(q, _BB // _L)
                tv = idx_v[buf, h, pl.ds(bq * _L, _L)] * D
                for c in range(D):
                    val = plsc.load_gather(tab_v, [tv + c])
                    o_v[buf, h, c, pl.ds(bq * _L, _L)] = val

            out_dma(blk, buf).start()

        out_dma(nblocks - 2, 0).wait()
        out_dma(nblocks - 1, 1).wait()

    out = emb_kernel(tab_flat, idx_t)
    return jnp.transpose(out, (2, 0, 1))


# speedup vs baseline: 1.7211x; 1.7211x over previous
import jax
import jax.numpy as jnp
from jax import lax
from jax.experimental import pallas as pl
from jax.experimental.pallas import tpu as pltpu
from jax.experimental.pallas import tpu_sc as plsc

_NC = 2
_NS = 16
_L = 16
_HB = 8
_BB = 128


def kernel(token_ids, table):
    B, H = token_ids.shape
    V, D = table.shape
    idx_t = token_ids.T
    tab_flat = table.reshape(-1)

    nw = _NC * _NS
    bblocks_per_w = B // _BB // nw
    hblocks = H // _HB
    nblocks = bblocks_per_w * hblocks

    mesh = plsc.VectorSubcoreMesh(core_axis_name="core", subcore_axis_name="subcore")

    @pl.kernel(
        out_type=jax.ShapeDtypeStruct((H, D, B), jnp.float32),
        mesh=mesh,
        scratch_types=[
            pltpu.VMEM((V * D,), jnp.float32),
            pltpu.VMEM((2, _HB, _BB), jnp.int32),
            pltpu.VMEM((2, _HB, D, _BB), jnp.float32),
            pltpu.SemaphoreType.DMA,
            pltpu.SemaphoreType.DMA,
            pltpu.SemaphoreType.DMA,
        ],
        compiler_params=pltpu.CompilerParams(
            use_tc_tiling_on_sc=True, needs_layout_passes=False
        ),
    )
    def emb_kernel(tab_hbm, idx_hbm, out_hbm, tab_v, idx_v, o_v, sem_t, sem_i, sem_o):
        wid = lax.axis_index("subcore") * _NC + lax.axis_index("core")
        b0 = wid * (bblocks_per_w * _BB)
        pltpu.async_copy(tab_hbm, tab_v, sem_t).wait()

        def blk_slices(blk):
            hb = lax.rem(blk, hblocks)
            bb = blk // hblocks
            return pl.ds(hb * _HB, _HB), pl.ds(b0 + bb * _BB, _BB)

        def idx_dma(blk, buf):
            hs, bs = blk_slices(blk)
            return pltpu.make_async_copy(idx_hbm.at[hs, bs], idx_v.at[buf], sem_i)

        def out_dma(blk, buf):
            hs, bs = blk_slices(blk)
            return pltpu.make_async_copy(
                o_v.at[buf], out_hbm.at[hs, :, bs], sem_o
            )

        idx_dma(0, 0).start()

        @pl.loop(0, nblocks)
        def _(blk):
            buf = lax.rem(blk, 2)
            nxt = 1 - buf

            @pl.when(blk + 1 < nblocks)
            def _():
                idx_dma(blk + 1, nxt).start()

            idx_dma(blk, buf).wait()

            @pl.when(blk >= 2)
            def _():
                out_dma(blk - 2, buf).wait()

            @pl.loop(0, _HB * (_BB // _L))
            def _(q):
                h = q // (_BB // _L)
                bq = lax.rem(q, _BB // _L)
                tv = idx_v[buf, h, pl.ds(bq * _L, _L)] * D
                vals = [plsc.load_gather(tab_v, [tv + c]) for c in range(D)]
                for c in range(D):
                    o_v[buf, h, c, pl.ds(bq * _L, _L)] = vals[c]

            out_dma(blk, buf).start()

        out_dma(nblocks - 2, 0).wait()
        out_dma(nblocks - 1, 1).wait()

    out = emb_kernel(tab_flat, idx_t)
    return jnp.transpose(out, (2, 0, 1))


# speedup vs baseline: 12.8814x; 7.4846x over previous
import jax
import jax.numpy as jnp
from jax import lax
from jax.experimental import pallas as pl
from jax.experimental.pallas import tpu as pltpu
from jax.experimental.pallas import tpu_sc as plsc

_NC = 2
_NS = 16
_L = 16
_HB = 8
_BB = 128


def kernel(token_ids, table):
    B, H = token_ids.shape
    V, D = table.shape
    idx_t = token_ids.T
    stride = D + 1
    tab_flat = jnp.pad(table, ((0, 0), (0, 1))).reshape(-1)

    nw = _NC * _NS
    bblocks_per_w = B // _BB // nw
    hblocks = H // _HB
    nblocks = bblocks_per_w * hblocks

    mesh = plsc.VectorSubcoreMesh(core_axis_name="core", subcore_axis_name="subcore")

    @pl.kernel(
        out_type=jax.ShapeDtypeStruct((H, D, B), jnp.float32),
        mesh=mesh,
        scratch_types=[
            pltpu.VMEM((V * stride,), jnp.float32),
            pltpu.VMEM((2, _HB, _BB), jnp.int32),
            pltpu.VMEM((2, _HB, D, _BB), jnp.float32),
            pltpu.SemaphoreType.DMA,
            pltpu.SemaphoreType.DMA,
            pltpu.SemaphoreType.DMA,
        ],
        compiler_params=pltpu.CompilerParams(
            use_tc_tiling_on_sc=True, needs_layout_passes=False
        ),
    )
    def emb_kernel(tab_hbm, idx_hbm, out_hbm, tab_v, idx_v, o_v, sem_t, sem_i, sem_o):
        wid = lax.axis_index("subcore") * _NC + lax.axis_index("core")
        b0 = wid * (bblocks_per_w * _BB)
        pltpu.async_copy(tab_hbm, tab_v, sem_t).wait()

        def blk_slices(blk):
            hb = lax.rem(blk, hblocks)
            bb = blk // hblocks
            return pl.ds(hb * _HB, _HB), pl.ds(b0 + bb * _BB, _BB)

        def idx_dma(blk, buf):
            hs, bs = blk_slices(blk)
            return pltpu.make_async_copy(idx_hbm.at[hs, bs], idx_v.at[buf], sem_i)

        def out_dma(blk, buf):
            hs, bs = blk_slices(blk)
            return pltpu.make_async_copy(
                o_v.at[buf], out_hbm.at[hs, :, bs], sem_o
            )

        idx_dma(0, 0).start()

        @pl.loop(0, nblocks)
        def _(blk):
            buf = lax.rem(blk, 2)
            nxt = 1 - buf

            @pl.when(blk + 1 < nblocks)
            def _():
                idx_dma(blk + 1, nxt).start()

            idx_dma(blk, buf).wait()

            @pl.when(blk >= 2)
            def _():
                out_dma(blk - 2, buf).wait()

            @plsc.parallel_loop(0, _HB * (_BB // _L))
            def _(q):
                h = q // (_BB // _L)
                bq = lax.rem(q, _BB // _L)
                tv = idx_v[buf, h, pl.ds(bq * _L, _L)] * stride
                vals = [plsc.load_gather(tab_v, [tv + c]) for c in range(D)]
                for c in range(D):
                    o_v[buf, h, c, pl.ds(bq * _L, _L)] = vals[c]

            out_dma(blk, buf).start()

        out_dma(nblocks - 2, 0).wait()
        out_dma(nblocks - 1, 1).wait()

    out = emb_kernel(tab_flat, idx_t)
    return jnp.transpose(out, (2, 0, 1))
